# baseline (speedup 1.0000x reference)
import functools

import jax
import jax.numpy as jnp
from jax import lax
from jax.experimental import pallas as pl
from jax.experimental.pallas import tpu as pltpu
from jax.experimental.pallas import tpu_sc as plsc

NC = 2
NS = 16
NW = NC * NS
GRP = 16


def _sc_gather(user_idx, item_idx, user_table, item_table):
    B = user_idx.shape[0]
    D = user_table.shape[1]
    bpw = B // NW
    uidx = user_idx.reshape(NW, bpw)
    iidx = item_idx.reshape(NW, bpw)
    mesh = plsc.VectorSubcoreMesh(core_axis_name="c", subcore_axis_name="s")

    @functools.partial(
        pl.kernel,
        out_type=(
            jax.ShapeDtypeStruct((NW, bpw, D), jnp.float32),
            jax.ShapeDtypeStruct((NW, bpw, D), jnp.float32),
        ),
        mesh=mesh,
        compiler_params=pltpu.CompilerParams(use_tc_tiling_on_sc=True,
                                             needs_layout_passes=False),
        scratch_types=[
            pltpu.VMEM((bpw,), jnp.int32),
            pltpu.VMEM((bpw,), jnp.int32),
            pltpu.SemaphoreType.DMA,
        ],
    )
    def gather_k(uidx_hbm, iidx_hbm, utab_hbm, itab_hbm, uout_hbm, iout_hbm,
                 uidx_v, iidx_v, sem):
        wid = lax.axis_index("s") * NC + lax.axis_index("c")
        pltpu.sync_copy(uidx_hbm.at[wid], uidx_v)
        pltpu.sync_copy(iidx_hbm.at[wid], iidx_v)
        lanes = lax.iota(jnp.int32, 16)

        def body(g, _):
            uvec = uidx_v[pl.ds(g * GRP, GRP)]
            ivec = iidx_v[pl.ds(g * GRP, GRP)]
            cps = []
            for l in range(GRP):
                ur = lax.reduce_max(jnp.where(lanes == l, uvec, 0), axes=(0,))
                cps.append(pltpu.async_copy(
                    utab_hbm.at[pl.ds(ur, 1), :],
                    uout_hbm.at[wid, pl.ds(g * GRP + l, 1), :], sem))
                ir = lax.reduce_max(jnp.where(lanes == l, ivec, 0), axes=(0,))
                cps.append(pltpu.async_copy(
                    itab_hbm.at[pl.ds(ir, 1), :],
                    iout_hbm.at[wid, pl.ds(g * GRP + l, 1), :], sem))
            for cp in cps:
                cp.wait()
            return 0

        lax.fori_loop(0, bpw // GRP, body, 0)

    u_rows, i_rows = gather_k(uidx, iidx, user_table, item_table)
    return u_rows.reshape(B, D), i_rows.reshape(B, D)


def _mlp_body(u_ref, i_ref, f_ref, ltab_ref, w1a_ref, w1b_ref, w1c_ref,
              b1_ref, w2_ref, b2_ref, uo_ref, io_ref):
    u = u_ref[...]
    n = jnp.sqrt(jnp.sum(u * u, axis=1, keepdims=True))
    uo_ref[...] = u / jnp.maximum(n, 1e-12)

    f = f_ref[...]
    lidx = jnp.clip(f[:, 2:3], 0.0, None).astype(jnp.int32)
    classes = lax.broadcasted_iota(jnp.int32, (1, ltab_ref.shape[0]), 1)
    onehot = (lidx == classes).astype(jnp.float32)
    lang = jnp.dot(onehot, ltab_ref[...],
                   preferred_element_type=jnp.float32)
    x = (jnp.dot(i_ref[...], w1a_ref[...], preferred_element_type=jnp.float32)
         + jnp.dot(lang, w1b_ref[...], preferred_element_type=jnp.float32)
         + f[:, 0:1] * w1c_ref[0:1, :] + f[:, 1:2] * w1c_ref[1:2, :]
         + b1_ref[...])
    h = jnp.maximum(x, 0.0)
    o = jnp.dot(h, w2_ref[...], preferred_element_type=jnp.float32) + b2_ref[...]
    n2 = jnp.sqrt(jnp.sum(o * o, axis=1, keepdims=True))
    io_ref[...] = o / jnp.maximum(n2, 1e-12)


def _tc_mlp(u_rows, i_rows, item_feats, lang_table, W1, b1, W2, b2):
    B, D = u_rows.shape
    L = lang_table.shape[0]
    E = lang_table.shape[1]
    BB = 2048
    grid = (B // BB,)
    w1a = W1[:, :D].T
    w1b = W1[:, D:D + E].T
    w1c = W1[:, D + E:].T
    b1r = b1.reshape(1, D)
    w2t = W2.T
    b2r = b2.reshape(1, D)
    full = lambda shape: pl.BlockSpec(shape, lambda b: (0, 0))
    return pl.pallas_call(
        _mlp_body,
        grid=grid,
        in_specs=[
            pl.BlockSpec((BB, D), lambda b: (b, 0)),
            pl.BlockSpec((BB, D), lambda b: (b, 0)),
            pl.BlockSpec((BB, 3), lambda b: (b, 0)),
            full((L, E)),
            full((D, D)),
            full((E, D)),
            full((2, D)),
            full((1, D)),
            full((D, D)),
            full((1, D)),
        ],
        out_specs=[
            pl.BlockSpec((BB, D), lambda b: (b, 0)),
            pl.BlockSpec((BB, D), lambda b: (b, 0)),
        ],
        out_shape=[
            jax.ShapeDtypeStruct((B, D), jnp.float32),
            jax.ShapeDtypeStruct((B, D), jnp.float32),
        ],
    )(u_rows, i_rows, item_feats, lang_table, w1a, w1b, w1c, b1r, w2t, b2r)


def kernel(user_idx, item_idx, item_feats, user_table, item_table, lang_table,
           W1, b1, W2, b2):
    u_rows, i_rows = _sc_gather(user_idx, item_idx, user_table, item_table)
    u, i = _tc_mlp(u_rows, i_rows, item_feats, lang_table, W1, b1, W2, b2)
    return (u, i)


# speedup vs baseline: 1.6485x; 1.6485x over previous
import functools

import jax
import jax.numpy as jnp
from jax import lax
from jax.experimental import pallas as pl
from jax.experimental.pallas import tpu as pltpu
from jax.experimental.pallas import tpu_sc as plsc

NC = 2
NS = 16
NW = NC * NS
GRP = 16
DEPTH = 4


def _sc_gather(user_idx, item_idx, user_table, item_table):
    B = user_idx.shape[0]
    D = user_table.shape[1]
    bpw = B // NW
    ngrp = bpw // GRP
    uidx = user_idx.reshape(NW, bpw)
    iidx = item_idx.reshape(NW, bpw)
    mesh = plsc.VectorSubcoreMesh(core_axis_name="c", subcore_axis_name="s")

    @functools.partial(
        pl.kernel,
        out_type=(
            jax.ShapeDtypeStruct((NW, bpw, D), jnp.float32),
            jax.ShapeDtypeStruct((NW, bpw, D), jnp.float32),
        ),
        mesh=mesh,
        compiler_params=pltpu.CompilerParams(use_tc_tiling_on_sc=True,
                                             needs_layout_passes=False),
        scratch_types=[
            pltpu.VMEM((bpw,), jnp.int32),
            pltpu.VMEM((bpw,), jnp.int32),
            pltpu.VMEM((bpw, D), jnp.float32),
            pltpu.SemaphoreType.DMA,
        ],
    )
    def gather_k(uidx_hbm, iidx_hbm, utab_hbm, itab_hbm, uout_hbm, iout_hbm,
                 uidx_v, iidx_v, rows_v, sem):
        wid = lax.axis_index("s") * NC + lax.axis_index("c")
        pltpu.sync_copy(uidx_hbm.at[wid], uidx_v)
        pltpu.sync_copy(iidx_hbm.at[wid], iidx_v)
        lanes = lax.iota(jnp.int32, 16)

        for tab_hbm, idx_v, out_hbm in ((utab_hbm, uidx_v, uout_hbm),
                                        (itab_hbm, iidx_v, iout_hbm)):
            def drain_one():
                pltpu.make_async_copy(
                    tab_hbm.at[pl.ds(0, 1), :],
                    rows_v.at[pl.ds(0, 1), :], sem).wait()

            def body(g, _):
                vec = idx_v[pl.ds(g * GRP, GRP)]
                for l in range(GRP):
                    r = lax.reduce_max(jnp.where(lanes == l, vec, 0), axes=(0,))
                    pltpu.async_copy(
                        tab_hbm.at[pl.ds(r, 1), :],
                        rows_v.at[pl.ds(g * GRP + l, 1), :], sem)

                @pl.when(g >= DEPTH)
                def _():
                    for _l in range(GRP):
                        drain_one()

                return 0

            lax.fori_loop(0, ngrp, body, 0)
            for _g in range(DEPTH):
                for _l in range(GRP):
                    drain_one()
            pltpu.sync_copy(rows_v, out_hbm.at[wid])

    u_rows, i_rows = gather_k(uidx, iidx, user_table, item_table)
    return u_rows.reshape(B, D), i_rows.reshape(B, D)


def _mlp_body(u_ref, i_ref, f_ref, ltab_ref, w1a_ref, w1b_ref, w1c_ref,
              b1_ref, w2_ref, b2_ref, utab_ref, uo_ref, io_ref):
    u = u_ref[...]
    n = jnp.sqrt(jnp.sum(u * u, axis=1, keepdims=True))
    uo_ref[...] = u / jnp.maximum(n, 1e-12)

    f = f_ref[...]
    lidx = jnp.clip(f[:, 2:3], 0.0, None).astype(jnp.int32)
    classes = lax.broadcasted_iota(jnp.int32, (1, ltab_ref.shape[0]), 1)
    onehot = (lidx == classes).astype(jnp.float32)
    lang = jnp.dot(onehot, ltab_ref[...],
                   preferred_element_type=jnp.float32)
    x = (jnp.dot(i_ref[...], w1a_ref[...], preferred_element_type=jnp.float32)
         + jnp.dot(lang, w1b_ref[...], preferred_element_type=jnp.float32)
         + f[:, 0:1] * w1c_ref[0:1, :] + f[:, 1:2] * w1c_ref[1:2, :]
         + b1_ref[...]
         + utab_ref[0:1, :] * 0.0)
    h = jnp.maximum(x, 0.0)
    o = jnp.dot(h, w2_ref[...], preferred_element_type=jnp.float32) + b2_ref[...]
    n2 = jnp.sqrt(jnp.sum(o * o, axis=1, keepdims=True))
    io_ref[...] = o / jnp.maximum(n2, 1e-12)


def _tc_mlp(u_rows, i_rows, item_feats, lang_table, W1, b1, W2, b2, user_table):
    B, D = u_rows.shape
    L = lang_table.shape[0]
    E = lang_table.shape[1]
    BB = 2048
    grid = (B // BB,)
    w1a = W1[:, :D].T
    w1b = W1[:, D:D + E].T
    w1c = W1[:, D + E:].T
    b1r = b1.reshape(1, D)
    w2t = W2.T
    b2r = b2.reshape(1, D)
    full = lambda shape: pl.BlockSpec(shape, lambda b: (0, 0))
    return pl.pallas_call(
        _mlp_body,
        grid=grid,
        in_specs=[
            pl.BlockSpec((BB, D), lambda b: (b, 0)),
            pl.BlockSpec((BB, D), lambda b: (b, 0)),
            pl.BlockSpec((BB, 3), lambda b: (b, 0)),
            full((L, E)),
            full((D, D)),
            full((E, D)),
            full((2, D)),
            full((1, D)),
            full((D, D)),
            full((1, D)),
            pl.BlockSpec((8, D), lambda b: (0, 0)),
        ],
        out_specs=[
            pl.BlockSpec((BB, D), lambda b: (b, 0)),
            pl.BlockSpec((BB, D), lambda b: (b, 0)),
        ],
        out_shape=[
            jax.ShapeDtypeStruct((B, D), jnp.float32),
            jax.ShapeDtypeStruct((B, D), jnp.float32),
        ],
    )(u_rows, i_rows, item_feats, lang_table, w1a, w1b, w1c, b1r, w2t, b2r,
      user_table)


def kernel(user_idx, item_idx, item_feats, user_table, item_table, lang_table,
           W1, b1, W2, b2):
    u_rows, i_rows = _sc_gather(user_idx, item_idx, user_table, item_table)
    u, i = _tc_mlp(u_rows, i_rows, item_feats, lang_table, W1, b1, W2, b2,
                   user_table)
    return (u, i)


# speedup vs baseline: 1.6590x; 1.0064x over previous
import functools

import jax
import jax.numpy as jnp
from jax import lax
from jax.experimental import pallas as pl
from jax.experimental.pallas import tpu as pltpu
from jax.experimental.pallas import tpu_sc as plsc

NC = 2
NS = 16
NW = NC * NS
GRP = 16
DEPTH = 4


def _sc_gather(user_idx, item_idx, user_table, item_table):
    B = user_idx.shape[0]
    D = user_table.shape[1]
    bpw = B // NW
    ngrp = bpw // GRP
    uidx = user_idx.reshape(NW, bpw)
    iidx = item_idx.reshape(NW, bpw)
    mesh = plsc.VectorSubcoreMesh(core_axis_name="c", subcore_axis_name="s")

    @functools.partial(
        pl.kernel,
        out_type=(
            jax.ShapeDtypeStruct((NW, bpw, D), jnp.float32),
            jax.ShapeDtypeStruct((NW, bpw, D), jnp.float32),
        ),
        mesh=mesh,
        compiler_params=pltpu.CompilerParams(use_tc_tiling_on_sc=True,
                                             needs_layout_passes=False),
        scratch_types=[
            pltpu.VMEM((bpw,), jnp.int32),
            pltpu.VMEM((bpw,), jnp.int32),
            pltpu.VMEM((bpw, D), jnp.float32),
            pltpu.SemaphoreType.DMA,
        ],
    )
    def gather_k(uidx_hbm, iidx_hbm, utab_hbm, itab_hbm, uout_hbm, iout_hbm,
                 uidx_v, iidx_v, rows_v, sem):
        wid = lax.axis_index("s") * NC + lax.axis_index("c")
        pltpu.sync_copy(uidx_hbm.at[wid], uidx_v)
        pltpu.sync_copy(iidx_hbm.at[wid], iidx_v)
        lanes = lax.iota(jnp.int32, 16)

        for tab_hbm, idx_v, out_hbm in ((utab_hbm, uidx_v, uout_hbm),
                                        (itab_hbm, iidx_v, iout_hbm)):
            def drain_one():
                pltpu.make_async_copy(
                    tab_hbm.at[pl.ds(0, 1), :],
                    rows_v.at[pl.ds(0, 1), :], sem).wait()

            def body(g, _):
                vec = idx_v[pl.ds(g * GRP, GRP)]
                for l in range(GRP):
                    r = lax.reduce_max(jnp.where(lanes == l, vec, 0), axes=(0,))
                    pltpu.async_copy(
                        tab_hbm.at[pl.ds(r, 1), :],
                        rows_v.at[pl.ds(g * GRP + l, 1), :], sem)

                @pl.when(g >= DEPTH)
                def _():
                    for _l in range(GRP):
                        drain_one()

                return 0

            lax.fori_loop(0, ngrp, body, 0)
            for _g in range(DEPTH):
                for _l in range(GRP):
                    drain_one()
            pltpu.sync_copy(rows_v, out_hbm.at[wid])

    u_rows, i_rows = gather_k(uidx, iidx, user_table, item_table)
    return u_rows.reshape(B, D), i_rows.reshape(B, D)


def _mlp_body(u_ref, i_ref, f_ref, ltab_ref, w1a_ref, w1b_ref, w1c_ref,
              b1_ref, w2_ref, b2_ref, uo_ref, io_ref):
    u = u_ref[...]
    n = jnp.sqrt(jnp.sum(u * u, axis=1, keepdims=True))
    uo_ref[...] = u / jnp.maximum(n, 1e-12)

    f = f_ref[...]
    lidx = jnp.clip(f[:, 2:3], 0.0, None).astype(jnp.int32)
    classes = lax.broadcasted_iota(jnp.int32, (1, ltab_ref.shape[0]), 1)
    onehot = (lidx == classes).astype(jnp.float32)
    lang = jnp.dot(onehot, ltab_ref[...],
                   preferred_element_type=jnp.float32)
    x = (jnp.dot(i_ref[...], w1a_ref[...], preferred_element_type=jnp.float32)
         + jnp.dot(lang, w1b_ref[...], preferred_element_type=jnp.float32)
         + f[:, 0:1] * w1c_ref[0:1, :] + f[:, 1:2] * w1c_ref[1:2, :]
         + b1_ref[...])
    h = jnp.maximum(x, 0.0)
    o = jnp.dot(h, w2_ref[...], preferred_element_type=jnp.float32) + b2_ref[...]
    n2 = jnp.sqrt(jnp.sum(o * o, axis=1, keepdims=True))
    io_ref[...] = o / jnp.maximum(n2, 1e-12)


def _tc_mlp(u_rows, i_rows, item_feats, lang_table, W1, b1, W2, b2):
    B, D = u_rows.shape
    L = lang_table.shape[0]
    E = lang_table.shape[1]
    BB = 2048
    grid = (B // BB,)
    w1a = W1[:, :D].T
    w1b = W1[:, D:D + E].T
    w1c = W1[:, D + E:].T
    b1r = b1.reshape(1, D)
    w2t = W2.T
    b2r = b2.reshape(1, D)
    full = lambda shape: pl.BlockSpec(shape, lambda b: (0, 0))
    return pl.pallas_call(
        _mlp_body,
        grid=grid,
        in_specs=[
            pl.BlockSpec((BB, D), lambda b: (b, 0)),
            pl.BlockSpec((BB, D), lambda b: (b, 0)),
            pl.BlockSpec((BB, 3), lambda b: (b, 0)),
            full((L, E)),
            full((D, D)),
            full((E, D)),
            full((2, D)),
            full((1, D)),
            full((D, D)),
            full((1, D)),
        ],
        out_specs=[
            pl.BlockSpec((BB, D), lambda b: (b, 0)),
            pl.BlockSpec((BB, D), lambda b: (b, 0)),
        ],
        out_shape=[
            jax.ShapeDtypeStruct((B, D), jnp.float32),
            jax.ShapeDtypeStruct((B, D), jnp.float32),
        ],
    )(u_rows, i_rows, item_feats, lang_table, w1a, w1b, w1c, b1r, w2t, b2r)


def _tc_identity(x):
    def body(x_ref, o_ref):
        pass
    return pl.pallas_call(
        body,
        in_specs=[pl.BlockSpec(memory_space=pl.ANY)],
        out_specs=pl.BlockSpec(memory_space=pl.ANY),
        out_shape=jax.ShapeDtypeStruct(x.shape, x.dtype),
        input_output_aliases={0: 0},
    )(x)


def kernel(user_idx, item_idx, item_feats, user_table, item_table, lang_table,
           W1, b1, W2, b2):
    u_tab = _tc_identity(user_table)
    u_rows, i_rows = _sc_gather(user_idx, item_idx, u_tab, item_table)
    u, i = _tc_mlp(u_rows, i_rows, item_feats, lang_table, W1, b1, W2, b2)
    return (u, i)
